# Initial kernel scaffold; baseline (speedup 1.0000x reference)
import jax
import jax.numpy as jnp
from jax.experimental import pallas as pl


def kernel(x, edge_index, W_proj, b_proj, W1, b1, b_n1, W_ffnn, b_ffnn):
    raise NotImplementedError("write your pallas kernel here")


# speedup vs baseline: 1.0246x; 1.0246x over previous
import jax
import jax.numpy as jnp
from jax.experimental import pallas as pl


def _ffnn_kernel(c_ref, w_ref, b_ref, o_ref):
    o_ref[...] = c_ref[...] @ w_ref[...] + b_ref[...]


def kernel(x, edge_index, W_proj, b_proj, W1, b1, b_n1, W_ffnn, b_ffnn):
    N = x.shape[0]
    h = x @ W_proj + b_proj
    t = jax.nn.sigmoid(h @ W1 + b1 + b_n1)
    src = edge_index[0]
    dst = edge_index[1]
    seg_ids = jnp.concatenate([dst, src])
    vals = t[jnp.concatenate([src, dst])]
    agg = jax.ops.segment_max(vals, seg_ids, num_segments=N)
    count = jnp.bincount(seg_ids, length=N)
    agg = jnp.where((count > 1)[:, None], agg, jnp.zeros_like(agg))
    combined = jnp.concatenate([h, agg], axis=1)
    out = pl.pallas_call(
        _ffnn_kernel,
        out_shape=jax.ShapeDtypeStruct((N, W_ffnn.shape[1]), jnp.float32),
    )(combined.astype(jnp.float32), W_ffnn.astype(jnp.float32),
      b_ffnn.astype(jnp.float32)[None, :])
    return out


# speedup vs baseline: 196.5633x; 191.8450x over previous
import functools

import jax
import jax.numpy as jnp
from jax import lax
from jax.experimental import pallas as pl
from jax.experimental.pallas import tpu as pltpu
from jax.experimental.pallas import tpu_sc as plsc

L = 16
GF = 8
GE = 4
F_PER = 4
CHUNK = 2000


def _front_body(x_ref, wp_ref, bp_ref, w1_ref, b1_ref, h_ref, tT_ref):
    xb = x_ref[...]
    hb = jnp.dot(xb, wp_ref[...], preferred_element_type=jnp.float32) + bp_ref[...]
    h_ref[...] = hb
    z = lax.dot_general(w1_ref[...], hb, (((0,), (1,)), ((), ())),
                        preferred_element_type=jnp.float32) + b1_ref[...]
    tT_ref[...] = 1.0 / (1.0 + jnp.exp(-z))


def _back_body(h_ref, agg_ref, cnt_ref, wt_ref, wb_ref, bf_ref, o_ref):
    a = jnp.max(agg_ref[...], axis=0)
    c = jnp.sum(cnt_ref[...], axis=0, dtype=jnp.int32)
    a = jnp.where((c > 1)[None, :], a, 0.0)
    o_ref[...] = (
        jnp.dot(h_ref[...], wt_ref[...], preferred_element_type=jnp.float32)
        + lax.dot_general(a, wb_ref[...], (((0,), (0,)), ((), ())),
                          preferred_element_type=jnp.float32)
        + bf_ref[...]
    )


def _sc_body(N, E, tT_hbm, src_hbm, dst_hbm, agg_out, cnt_out,
             t_v, agg_v, own_v, cnt_v, sb0, db0, sb1, db1, sem):
    i32 = jnp.int32
    cid = lax.axis_index("c").astype(i32)
    sid = lax.axis_index("s").astype(i32)
    wid = sid * i32(2) + cid
    gf = wid // i32(GE)
    ge = wid % i32(GE)
    epg = E // GE
    base = ge * i32(epg)
    nchunk = epg // CHUNK

    lane = lax.iota(jnp.int32, L)
    ones_i = jnp.ones((L,), jnp.int32)
    fvecs = [jnp.full((L,), f, jnp.int32) for f in range(F_PER)]

    pltpu.sync_copy(tT_hbm.at[pl.ds(gf * i32(F_PER), F_PER)], t_v)

    def _z(i, carry):
        o = i * i32(L)
        z16f = jnp.zeros((L,), jnp.float32)
        for f in range(F_PER):
            agg_v[i32(f), pl.ds(o, L)] = z16f
        cnt_v[pl.ds(o, L)] = jnp.zeros((L,), jnp.int32)
        return carry
    lax.fori_loop(i32(0), i32(N // L), _z, i32(0))

    def _direction(avec, bvec):
        @pl.when(gf == i32(0))
        def _():
            plsc.addupdate_scatter(cnt_v, [avec], ones_i)
        plsc.store_scatter(own_v, [avec], lane)
        rb = plsc.load_gather(own_v, [avec])
        win = rb == lane
        for f in range(F_PER):
            val = plsc.load_gather(t_v, [fvecs[f], bvec])
            cur = plsc.load_gather(agg_v, [fvecs[f], avec])
            plsc.store_scatter(agg_v, [fvecs[f], avec],
                               jnp.maximum(cur, val), mask=win)
        rem = jnp.where(win, jnp.zeros((L,), jnp.int32), ones_i)

        def _wcond(r):
            return jnp.max(r) > i32(0)

        def _wbody(r):
            m = r > 0
            plsc.store_scatter(own_v, [avec], lane, mask=m)
            rb2 = plsc.load_gather(own_v, [avec])
            win2 = m & (rb2 == lane)
            for f in range(F_PER):
                val = plsc.load_gather(t_v, [fvecs[f], bvec])
                cur = plsc.load_gather(agg_v, [fvecs[f], avec])
                plsc.store_scatter(agg_v, [fvecs[f], avec],
                                   jnp.maximum(cur, val), mask=win2)
            return jnp.where(win2, jnp.zeros((L,), jnp.int32), r)

        lax.while_loop(_wcond, _wbody, rem)

    bufs = ((sb0, db0), (sb1, db1))

    def _process(b):
        sb, db = bufs[b]

        def _grp(k, carry):
            o = k * i32(L)
            svec = sb[pl.ds(o, L)]
            dvec = db[pl.ds(o, L)]
            _direction(dvec, svec)
            _direction(svec, dvec)
            return carry
        lax.fori_loop(i32(0), i32(CHUNK // L), _grp, i32(0))

    def _start(ci, b):
        off = base + ci * i32(CHUNK)
        sb, db = bufs[b]
        pltpu.async_copy(src_hbm.at[pl.ds(off, CHUNK)], sb, sem.at[i32(b), i32(0)])
        pltpu.async_copy(dst_hbm.at[pl.ds(off, CHUNK)], db, sem.at[i32(b), i32(1)])

    def _wait(b):
        sb, db = bufs[b]
        pltpu.make_async_copy(src_hbm.at[pl.ds(base, CHUNK)], sb,
                              sem.at[i32(b), i32(0)]).wait()
        pltpu.make_async_copy(dst_hbm.at[pl.ds(base, CHUNK)], db,
                              sem.at[i32(b), i32(1)]).wait()

    _start(i32(0), 0)
    _start(i32(1), 1)

    def _outer(ci2, carry):
        for b in range(2):
            ci = ci2 * i32(2) + i32(b)
            _wait(b)
            _process(b)
            nxt = ci + i32(2)

            @pl.when(nxt < i32(nchunk))
            def _():
                _start(nxt, b)
        return carry
    lax.fori_loop(i32(0), i32(nchunk // 2), _outer, i32(0))

    pltpu.sync_copy(agg_v, agg_out.at[ge, pl.ds(gf * i32(F_PER), F_PER)])

    @pl.when(gf == i32(0))
    def _():
        pltpu.sync_copy(cnt_v, cnt_out.at[ge])


def kernel(x, edge_index, W_proj, b_proj, W1, b1, b_n1, W_ffnn, b_ffnn):
    N, D = x.shape
    H = W_proj.shape[1]
    E = edge_index.shape[1]
    x = x.astype(jnp.float32)
    edges = edge_index.astype(jnp.int32)

    h, t_T = pl.pallas_call(
        _front_body,
        out_shape=[
            jax.ShapeDtypeStruct((N, H), jnp.float32),
            jax.ShapeDtypeStruct((H, N), jnp.float32),
        ],
    )(x, W_proj.astype(jnp.float32), b_proj.astype(jnp.float32)[None, :],
      W1.astype(jnp.float32),
      (b1 + b_n1).astype(jnp.float32)[:, None])

    mesh = plsc.VectorSubcoreMesh(core_axis_name="c", subcore_axis_name="s",
                                  num_cores=2, num_subcores=16)
    sc = pl.kernel(
        functools.partial(_sc_body, N, E),
        out_type=[
            jax.ShapeDtypeStruct((GE, H, N), jnp.float32),
            jax.ShapeDtypeStruct((GE, N), jnp.int32),
        ],
        mesh=mesh,
        compiler_params=pltpu.CompilerParams(needs_layout_passes=False),
        scratch_types=[
            pltpu.VMEM((F_PER, N), jnp.float32),
            pltpu.VMEM((F_PER, N), jnp.float32),
            pltpu.VMEM((N,), jnp.int32),
            pltpu.VMEM((N,), jnp.int32),
            pltpu.VMEM((CHUNK,), jnp.int32),
            pltpu.VMEM((CHUNK,), jnp.int32),
            pltpu.VMEM((CHUNK,), jnp.int32),
            pltpu.VMEM((CHUNK,), jnp.int32),
            pltpu.SemaphoreType.DMA((2, 2)),
        ],
    )
    agg_parts, counts = sc(t_T, edges[0], edges[1])

    out = pl.pallas_call(
        _back_body,
        out_shape=jax.ShapeDtypeStruct((N, H), jnp.float32),
    )(h, agg_parts, counts,
      W_ffnn[:H].astype(jnp.float32), W_ffnn[H:].astype(jnp.float32),
      b_ffnn.astype(jnp.float32)[None, :])
    return out


# speedup vs baseline: 197.4608x; 1.0046x over previous
import functools

import jax
import jax.numpy as jnp
from jax import lax
from jax.experimental import pallas as pl
from jax.experimental.pallas import tpu as pltpu
from jax.experimental.pallas import tpu_sc as plsc

L = 16
GF = 8
GE = 4
F_PER = 4
CHUNK = 2000


def _front_body(x_ref, wp_ref, bp_ref, w1_ref, b1_ref, h_ref, tT_ref):
    xb = x_ref[...]
    hb = jnp.dot(xb, wp_ref[...], preferred_element_type=jnp.float32) + bp_ref[...]
    h_ref[...] = hb
    z = lax.dot_general(w1_ref[...], hb, (((0,), (1,)), ((), ())),
                        preferred_element_type=jnp.float32) + b1_ref[...]
    tT_ref[...] = 1.0 / (1.0 + jnp.exp(-z))


def _back_body(h_ref, agg_ref, cnt_ref, wt_ref, wb_ref, bf_ref, o_ref):
    a = jnp.max(agg_ref[...], axis=0)
    c = jnp.sum(cnt_ref[...], axis=0, dtype=jnp.int32)
    a = jnp.where((c > 1)[None, :], a, 0.0)
    o_ref[...] = (
        jnp.dot(h_ref[...], wt_ref[...], preferred_element_type=jnp.float32)
        + lax.dot_general(a, wb_ref[...], (((0,), (0,)), ((), ())),
                          preferred_element_type=jnp.float32)
        + bf_ref[...]
    )


def _sc_body(N, E, tT_hbm, src_hbm, dst_hbm, agg_out, cnt_out,
             t0, t1, t2, t3, a0, a1, a2, a3,
             own_a, own_b, cnt_v, sb0, db0, sb1, db1, sem):
    i32 = jnp.int32
    cid = lax.axis_index("c").astype(i32)
    sid = lax.axis_index("s").astype(i32)
    wid = sid * i32(2) + cid
    gf = wid // i32(GE)
    ge = wid % i32(GE)
    epg = E // GE
    base = ge * i32(epg)
    nchunk = epg // CHUNK

    lane = lax.iota(jnp.int32, L)
    ones_i = jnp.ones((L,), jnp.int32)
    t_f = [t0, t1, t2, t3]
    agg_f = [a0, a1, a2, a3]

    for f in range(F_PER):
        pltpu.sync_copy(tT_hbm.at[gf * i32(F_PER) + i32(f)], t_f[f])

    def _z(i, carry):
        o = i * i32(L)
        z16f = jnp.zeros((L,), jnp.float32)
        for f in range(F_PER):
            agg_f[f][pl.ds(o, L)] = z16f
        cnt_v[pl.ds(o, L)] = jnp.zeros((L,), jnp.int32)
        return carry
    lax.fori_loop(i32(0), i32(N // L), _z, i32(0))

    def _direction(avec, bvec, own_v):
        @pl.when(gf == i32(0))
        def _():
            plsc.addupdate_scatter(cnt_v, [avec], ones_i)
        plsc.store_scatter(own_v, [avec], lane)
        rb = plsc.load_gather(own_v, [avec])
        win = rb == lane
        for f in range(F_PER):
            val = plsc.load_gather(t_f[f], [bvec])
            cur = plsc.load_gather(agg_f[f], [avec])
            plsc.store_scatter(agg_f[f], [avec],
                               jnp.maximum(cur, val), mask=win)
        rem = jnp.where(win, jnp.zeros((L,), jnp.int32), ones_i)

        def _wcond(r):
            return jnp.max(r) > i32(0)

        def _wbody(r):
            m = r > 0
            plsc.store_scatter(own_v, [avec], lane, mask=m)
            rb2 = plsc.load_gather(own_v, [avec])
            win2 = m & (rb2 == lane)
            for f in range(F_PER):
                val = plsc.load_gather(t_f[f], [bvec])
                cur = plsc.load_gather(agg_f[f], [avec])
                plsc.store_scatter(agg_f[f], [avec],
                                   jnp.maximum(cur, val), mask=win2)
            return jnp.where(win2, jnp.zeros((L,), jnp.int32), r)

        lax.while_loop(_wcond, _wbody, rem)

    bufs = ((sb0, db0), (sb1, db1))

    def _process(b):
        sb, db = bufs[b]

        def _grp(k, carry):
            o = k * i32(L)
            svec = sb[pl.ds(o, L)]
            dvec = db[pl.ds(o, L)]
            _direction(dvec, svec, own_a)
            _direction(svec, dvec, own_b)
            return carry
        lax.fori_loop(i32(0), i32(CHUNK // L), _grp, i32(0))

    def _start(ci, b):
        off = base + ci * i32(CHUNK)
        sb, db = bufs[b]
        pltpu.async_copy(src_hbm.at[pl.ds(off, CHUNK)], sb, sem.at[i32(b), i32(0)])
        pltpu.async_copy(dst_hbm.at[pl.ds(off, CHUNK)], db, sem.at[i32(b), i32(1)])

    def _wait(b):
        sb, db = bufs[b]
        pltpu.make_async_copy(src_hbm.at[pl.ds(base, CHUNK)], sb,
                              sem.at[i32(b), i32(0)]).wait()
        pltpu.make_async_copy(dst_hbm.at[pl.ds(base, CHUNK)], db,
                              sem.at[i32(b), i32(1)]).wait()

    _start(i32(0), 0)
    _start(i32(1), 1)

    def _outer(ci2, carry):
        for b in range(2):
            ci = ci2 * i32(2) + i32(b)
            _wait(b)
            _process(b)
            nxt = ci + i32(2)

            @pl.when(nxt < i32(nchunk))
            def _():
                _start(nxt, b)
        return carry
    lax.fori_loop(i32(0), i32(nchunk // 2), _outer, i32(0))

    for f in range(F_PER):
        pltpu.sync_copy(agg_f[f], agg_out.at[ge, gf * i32(F_PER) + i32(f)])

    @pl.when(gf == i32(0))
    def _():
        pltpu.sync_copy(cnt_v, cnt_out.at[ge])


def kernel(x, edge_index, W_proj, b_proj, W1, b1, b_n1, W_ffnn, b_ffnn):
    N, D = x.shape
    H = W_proj.shape[1]
    E = edge_index.shape[1]
    x = x.astype(jnp.float32)
    edges = edge_index.astype(jnp.int32)

    h, t_T = pl.pallas_call(
        _front_body,
        out_shape=[
            jax.ShapeDtypeStruct((N, H), jnp.float32),
            jax.ShapeDtypeStruct((H, N), jnp.float32),
        ],
    )(x, W_proj.astype(jnp.float32), b_proj.astype(jnp.float32)[None, :],
      W1.astype(jnp.float32),
      (b1 + b_n1).astype(jnp.float32)[:, None])

    mesh = plsc.VectorSubcoreMesh(core_axis_name="c", subcore_axis_name="s",
                                  num_cores=2, num_subcores=16)
    sc = pl.kernel(
        functools.partial(_sc_body, N, E),
        out_type=[
            jax.ShapeDtypeStruct((GE, H, N), jnp.float32),
            jax.ShapeDtypeStruct((GE, N), jnp.int32),
        ],
        mesh=mesh,
        compiler_params=pltpu.CompilerParams(needs_layout_passes=False),
        scratch_types=[
            pltpu.VMEM((N,), jnp.float32),
            pltpu.VMEM((N,), jnp.float32),
            pltpu.VMEM((N,), jnp.float32),
            pltpu.VMEM((N,), jnp.float32),
            pltpu.VMEM((N,), jnp.float32),
            pltpu.VMEM((N,), jnp.float32),
            pltpu.VMEM((N,), jnp.float32),
            pltpu.VMEM((N,), jnp.float32),
            pltpu.VMEM((N,), jnp.int32),
            pltpu.VMEM((N,), jnp.int32),
            pltpu.VMEM((N,), jnp.int32),
            pltpu.VMEM((CHUNK,), jnp.int32),
            pltpu.VMEM((CHUNK,), jnp.int32),
            pltpu.VMEM((CHUNK,), jnp.int32),
            pltpu.VMEM((CHUNK,), jnp.int32),
            pltpu.SemaphoreType.DMA((2, 2)),
        ],
    )
    agg_parts, counts = sc(t_T, edges[0], edges[1])

    out = pl.pallas_call(
        _back_body,
        out_shape=jax.ShapeDtypeStruct((N, H), jnp.float32),
    )(h, agg_parts, counts,
      W_ffnn[:H].astype(jnp.float32), W_ffnn[H:].astype(jnp.float32),
      b_ffnn.astype(jnp.float32)[None, :])
    return out


# speedup vs baseline: 202.7559x; 1.0268x over previous
import functools

import jax
import jax.numpy as jnp
from jax import lax
from jax.experimental import pallas as pl
from jax.experimental.pallas import tpu as pltpu
from jax.experimental.pallas import tpu_sc as plsc

L = 16
GF = 8
GE = 4
F_PER = 4
CHUNK = 2000


def _front_body(x_ref, wp_ref, bp_ref, w1_ref, b1_ref, h_ref, tT_ref):
    xb = x_ref[...]
    hb = jnp.dot(xb, wp_ref[...], preferred_element_type=jnp.float32) + bp_ref[...]
    h_ref[...] = hb
    z = lax.dot_general(w1_ref[...], hb, (((0,), (1,)), ((), ())),
                        preferred_element_type=jnp.float32) + b1_ref[...]
    tT_ref[...] = 1.0 / (1.0 + jnp.exp(-z))


def _back_body(h_ref, agg_ref, cnt_ref, wt_ref, wb_ref, bf_ref, o_ref):
    a = jnp.max(agg_ref[...], axis=0)
    c = jnp.sum(cnt_ref[...], axis=0, dtype=jnp.int32)
    a = jnp.where((c > 1)[None, :], a, 0.0)
    o_ref[...] = (
        jnp.dot(h_ref[...], wt_ref[...], preferred_element_type=jnp.float32)
        + lax.dot_general(a, wb_ref[...], (((0,), (0,)), ((), ())),
                          preferred_element_type=jnp.float32)
        + bf_ref[...]
    )


def _sc_body(N, E, tT_hbm, src_hbm, dst_hbm, agg_out, cnt_out,
             t0, t1, t2, t3, a0, a1, a2, a3,
             own_a, own_b, cnt_v, sb0, db0, sb1, db1, sem):
    i32 = jnp.int32
    cid = lax.axis_index("c").astype(i32)
    sid = lax.axis_index("s").astype(i32)
    wid = sid * i32(2) + cid
    gf = wid // i32(GE)
    ge = wid % i32(GE)
    epg = E // GE
    base = ge * i32(epg)
    nchunk = epg // CHUNK

    lane = lax.iota(jnp.int32, L)
    ones_i = jnp.ones((L,), jnp.int32)
    t_f = [t0, t1, t2, t3]
    agg_f = [a0, a1, a2, a3]

    for f in range(F_PER):
        pltpu.sync_copy(tT_hbm.at[gf * i32(F_PER) + i32(f)], t_f[f])

    def _z(i, carry):
        o = i * i32(L)
        z16f = jnp.zeros((L,), jnp.float32)
        for f in range(F_PER):
            agg_f[f][pl.ds(o, L)] = z16f
        cnt_v[pl.ds(o, L)] = jnp.zeros((L,), jnp.int32)
        return carry
    lax.fori_loop(i32(0), i32(N // L), _z, i32(0))

    def _direction(avec, bvec, own_v):
        @pl.when(gf == i32(0))
        def _():
            plsc.addupdate_scatter(cnt_v, [avec], ones_i)
        plsc.store_scatter(own_v, [avec], lane)
        rb = plsc.load_gather(own_v, [avec])
        win = rb == lane
        for f in range(F_PER):
            val = plsc.load_gather(t_f[f], [bvec])
            cur = plsc.load_gather(agg_f[f], [avec])
            plsc.store_scatter(agg_f[f], [avec],
                               jnp.maximum(cur, val), mask=win)
        rem = jnp.where(win, jnp.zeros((L,), jnp.int32), ones_i)

        def _wcond(r):
            return jnp.max(r) > i32(0)

        def _wbody(r):
            m = r > 0
            plsc.store_scatter(own_v, [avec], lane, mask=m)
            rb2 = plsc.load_gather(own_v, [avec])
            win2 = m & (rb2 == lane)
            for f in range(F_PER):
                val = plsc.load_gather(t_f[f], [bvec])
                cur = plsc.load_gather(agg_f[f], [avec])
                plsc.store_scatter(agg_f[f], [avec],
                                   jnp.maximum(cur, val), mask=win2)
            return jnp.where(win2, jnp.zeros((L,), jnp.int32), r)

        lax.while_loop(_wcond, _wbody, rem)

    bufs = ((sb0, db0), (sb1, db1))

    def _process(b):
        sb, db = bufs[b]

        def _grp(k, carry):
            o = k * i32(2 * L)
            for u in range(2):
                ou = o + i32(u * L)
                svec = sb[pl.ds(ou, L)]
                dvec = db[pl.ds(ou, L)]
                _direction(dvec, svec, own_a)
                _direction(svec, dvec, own_b)
            return carry
        lax.fori_loop(i32(0), i32(CHUNK // L // 2), _grp, i32(0))

    def _start(ci, b):
        off = base + ci * i32(CHUNK)
        sb, db = bufs[b]
        pltpu.async_copy(src_hbm.at[pl.ds(off, CHUNK)], sb, sem.at[i32(b), i32(0)])
        pltpu.async_copy(dst_hbm.at[pl.ds(off, CHUNK)], db, sem.at[i32(b), i32(1)])

    def _wait(b):
        sb, db = bufs[b]
        pltpu.make_async_copy(src_hbm.at[pl.ds(base, CHUNK)], sb,
                              sem.at[i32(b), i32(0)]).wait()
        pltpu.make_async_copy(dst_hbm.at[pl.ds(base, CHUNK)], db,
                              sem.at[i32(b), i32(1)]).wait()

    _start(i32(0), 0)
    _start(i32(1), 1)

    def _outer(ci2, carry):
        for b in range(2):
            ci = ci2 * i32(2) + i32(b)
            _wait(b)
            _process(b)
            nxt = ci + i32(2)

            @pl.when(nxt < i32(nchunk))
            def _():
                _start(nxt, b)
        return carry
    lax.fori_loop(i32(0), i32(nchunk // 2), _outer, i32(0))

    for f in range(F_PER):
        pltpu.sync_copy(agg_f[f], agg_out.at[ge, gf * i32(F_PER) + i32(f)])

    @pl.when(gf == i32(0))
    def _():
        pltpu.sync_copy(cnt_v, cnt_out.at[ge])


def kernel(x, edge_index, W_proj, b_proj, W1, b1, b_n1, W_ffnn, b_ffnn):
    N, D = x.shape
    H = W_proj.shape[1]
    E = edge_index.shape[1]
    x = x.astype(jnp.float32)
    edges = edge_index.astype(jnp.int32)

    h, t_T = pl.pallas_call(
        _front_body,
        out_shape=[
            jax.ShapeDtypeStruct((N, H), jnp.float32),
            jax.ShapeDtypeStruct((H, N), jnp.float32),
        ],
    )(x, W_proj.astype(jnp.float32), b_proj.astype(jnp.float32)[None, :],
      W1.astype(jnp.float32),
      (b1 + b_n1).astype(jnp.float32)[:, None])

    mesh = plsc.VectorSubcoreMesh(core_axis_name="c", subcore_axis_name="s",
                                  num_cores=2, num_subcores=16)
    sc = pl.kernel(
        functools.partial(_sc_body, N, E),
        out_type=[
            jax.ShapeDtypeStruct((GE, H, N), jnp.float32),
            jax.ShapeDtypeStruct((GE, N), jnp.int32),
        ],
        mesh=mesh,
        compiler_params=pltpu.CompilerParams(needs_layout_passes=False),
        scratch_types=[
            pltpu.VMEM((N,), jnp.float32),
            pltpu.VMEM((N,), jnp.float32),
            pltpu.VMEM((N,), jnp.float32),
            pltpu.VMEM((N,), jnp.float32),
            pltpu.VMEM((N,), jnp.float32),
            pltpu.VMEM((N,), jnp.float32),
            pltpu.VMEM((N,), jnp.float32),
            pltpu.VMEM((N,), jnp.float32),
            pltpu.VMEM((N,), jnp.int32),
            pltpu.VMEM((N,), jnp.int32),
            pltpu.VMEM((N,), jnp.int32),
            pltpu.VMEM((CHUNK,), jnp.int32),
            pltpu.VMEM((CHUNK,), jnp.int32),
            pltpu.VMEM((CHUNK,), jnp.int32),
            pltpu.VMEM((CHUNK,), jnp.int32),
            pltpu.SemaphoreType.DMA((2, 2)),
        ],
    )
    agg_parts, counts = sc(t_T, edges[0], edges[1])

    out = pl.pallas_call(
        _back_body,
        out_shape=jax.ShapeDtypeStruct((N, H), jnp.float32),
    )(h, agg_parts, counts,
      W_ffnn[:H].astype(jnp.float32), W_ffnn[H:].astype(jnp.float32),
      b_ffnn.astype(jnp.float32)[None, :])
    return out


# speedup vs baseline: 331.1965x; 1.6335x over previous
import functools

import jax
import jax.numpy as jnp
from jax import lax
from jax.experimental import pallas as pl
from jax.experimental.pallas import tpu as pltpu
from jax.experimental.pallas import tpu_sc as plsc

L = 16
GFg = 4
GE = 8
W_PER = 4
CHUNK = 2000


def _front_body(x_ref, wp_ref, bp_ref, w1e_ref, w1o_ref, b1e_ref, b1o_ref,
                h_ref, tp_ref):
    xb = x_ref[...]
    hb = jnp.dot(xb, wp_ref[...], preferred_element_type=jnp.float32) + bp_ref[...]
    h_ref[...] = hb
    ze = lax.dot_general(w1e_ref[...], hb, (((0,), (1,)), ((), ())),
                         preferred_element_type=jnp.float32) + b1e_ref[...]
    zo = lax.dot_general(w1o_ref[...], hb, (((0,), (1,)), ((), ())),
                         preferred_element_type=jnp.float32) + b1o_ref[...]
    se = 1.0 / (1.0 + jnp.exp(-ze))
    so = 1.0 / (1.0 + jnp.exp(-zo))
    ue = lax.bitcast_convert_type(se.astype(jnp.bfloat16), jnp.uint16)
    uo = lax.bitcast_convert_type(so.astype(jnp.bfloat16), jnp.uint16)
    word = ue.astype(jnp.uint32) | (uo.astype(jnp.uint32) << 16)
    tp_ref[...] = lax.bitcast_convert_type(word, jnp.int32)


def _back_body(h_ref, agg_ref, cnt_ref, wt_ref, wbe_ref, wbo_ref, bf_ref,
               o_ref):
    w = lax.bitcast_convert_type(agg_ref[...], jnp.uint32)
    lo = lax.bitcast_convert_type(w << 16, jnp.float32)
    hi = lax.bitcast_convert_type(w & jnp.uint32(0xFFFF0000), jnp.float32)
    alo = jnp.max(lo, axis=0)
    ahi = jnp.max(hi, axis=0)
    c = jnp.sum(cnt_ref[...], axis=0, dtype=jnp.int32)
    msk = (c > 1)[None, :]
    alo = jnp.where(msk, alo, 0.0)
    ahi = jnp.where(msk, ahi, 0.0)
    o_ref[...] = (
        jnp.dot(h_ref[...], wt_ref[...], preferred_element_type=jnp.float32)
        + lax.dot_general(alo, wbe_ref[...], (((0,), (0,)), ((), ())),
                          preferred_element_type=jnp.float32)
        + lax.dot_general(ahi, wbo_ref[...], (((0,), (0,)), ((), ())),
                          preferred_element_type=jnp.float32)
        + bf_ref[...]
    )


def _sc_body(N, E, tp_hbm, src_hbm, dst_hbm, agg_out, cnt_out,
             t0, t1, t2, t3, a0, a1, a2, a3,
             own_a, own_b, cnt_v, sb0, db0, sb1, db1, sem):
    i32 = jnp.int32
    cid = lax.axis_index("c").astype(i32)
    sid = lax.axis_index("s").astype(i32)
    wid = sid * i32(2) + cid
    gf = wid // i32(GE)
    ge = wid % i32(GE)
    epg = E // GE
    base = ge * i32(epg)
    nchunk = epg // CHUNK

    lane = lax.iota(jnp.int32, L)
    ones_i = jnp.ones((L,), jnp.int32)
    t_w = [t0, t1, t2, t3]
    agg_w = [a0, a1, a2, a3]

    for w in range(W_PER):
        pltpu.sync_copy(tp_hbm.at[gf * i32(W_PER) + i32(w)], t_w[w])

    def _z(i, carry):
        o = i * i32(L)
        z16i = jnp.zeros((L,), jnp.int32)
        for w in range(W_PER):
            agg_w[w][pl.ds(o, L)] = z16i
        cnt_v[pl.ds(o, L)] = z16i
        return carry
    lax.fori_loop(i32(0), i32(N // L), _z, i32(0))

    def _rmw(w, avec, bvec, mask):
        val = plsc.load_gather(t_w[w], [bvec])
        cur = plsc.load_gather(agg_w[w], [avec])
        mx = jnp.maximum(plsc.bitcast(val, jnp.bfloat16),
                         plsc.bitcast(cur, jnp.bfloat16))
        plsc.store_scatter(agg_w[w], [avec], plsc.bitcast(mx, jnp.int32),
                           mask=mask)

    def _direction(avec, bvec, own_v):
        @pl.when(gf == i32(0))
        def _():
            plsc.addupdate_scatter(cnt_v, [avec], ones_i)
        plsc.store_scatter(own_v, [avec], lane)
        rb = plsc.load_gather(own_v, [avec])
        win = rb == lane
        for w in range(W_PER):
            _rmw(w, avec, bvec, win)
        rem = jnp.where(win, jnp.zeros((L,), jnp.int32), ones_i)

        def _wcond(r):
            return jnp.max(r) > i32(0)

        def _wbody(r):
            m = r > 0
            plsc.store_scatter(own_v, [avec], lane, mask=m)
            rb2 = plsc.load_gather(own_v, [avec])
            win2 = m & (rb2 == lane)
            for w in range(W_PER):
                _rmw(w, avec, bvec, win2)
            return jnp.where(win2, jnp.zeros((L,), jnp.int32), r)

        lax.while_loop(_wcond, _wbody, rem)

    bufs = ((sb0, db0), (sb1, db1))

    def _process(b):
        sb, db = bufs[b]

        def _grp(k, carry):
            o = k * i32(L)
            svec = sb[pl.ds(o, L)]
            dvec = db[pl.ds(o, L)]
            _direction(dvec, svec, own_a)
            _direction(svec, dvec, own_b)
            return carry
        lax.fori_loop(i32(0), i32(CHUNK // L), _grp, i32(0))

    def _start(ci, b):
        off = base + ci * i32(CHUNK)
        sb, db = bufs[b]
        pltpu.async_copy(src_hbm.at[pl.ds(off, CHUNK)], sb, sem.at[i32(b), i32(0)])
        pltpu.async_copy(dst_hbm.at[pl.ds(off, CHUNK)], db, sem.at[i32(b), i32(1)])

    def _wait(b):
        sb, db = bufs[b]
        pltpu.make_async_copy(src_hbm.at[pl.ds(base, CHUNK)], sb,
                              sem.at[i32(b), i32(0)]).wait()
        pltpu.make_async_copy(dst_hbm.at[pl.ds(base, CHUNK)], db,
                              sem.at[i32(b), i32(1)]).wait()

    _start(i32(0), 0)
    _start(i32(1), 1)

    def _outer(ci2, carry):
        for b in range(2):
            ci = ci2 * i32(2) + i32(b)
            _wait(b)
            _process(b)
            nxt = ci + i32(2)

            @pl.when(nxt < i32(nchunk))
            def _():
                _start(nxt, b)
        return carry
    lax.fori_loop(i32(0), i32(nchunk // 2), _outer, i32(0))

    for w in range(W_PER):
        pltpu.sync_copy(agg_w[w], agg_out.at[ge, gf * i32(W_PER) + i32(w)])

    @pl.when(gf == i32(0))
    def _():
        pltpu.sync_copy(cnt_v, cnt_out.at[ge])


def kernel(x, edge_index, W_proj, b_proj, W1, b1, b_n1, W_ffnn, b_ffnn):
    N, D = x.shape
    H = W_proj.shape[1]
    E = edge_index.shape[1]
    NP = H // 2
    x = x.astype(jnp.float32)
    edges = edge_index.astype(jnp.int32)
    W1f = W1.astype(jnp.float32)
    b1f = (b1 + b_n1).astype(jnp.float32)
    Wb = W_ffnn[H:].astype(jnp.float32)

    h, t_pack = pl.pallas_call(
        _front_body,
        out_shape=[
            jax.ShapeDtypeStruct((N, H), jnp.float32),
            jax.ShapeDtypeStruct((NP, N), jnp.int32),
        ],
    )(x, W_proj.astype(jnp.float32), b_proj.astype(jnp.float32)[None, :],
      W1f[:, 0::2], W1f[:, 1::2], b1f[0::2][:, None], b1f[1::2][:, None])

    mesh = plsc.VectorSubcoreMesh(core_axis_name="c", subcore_axis_name="s",
                                  num_cores=2, num_subcores=16)
    sc = pl.kernel(
        functools.partial(_sc_body, N, E),
        out_type=[
            jax.ShapeDtypeStruct((GE, NP, N), jnp.int32),
            jax.ShapeDtypeStruct((GE, N), jnp.int32),
        ],
        mesh=mesh,
        compiler_params=pltpu.CompilerParams(needs_layout_passes=False),
        scratch_types=[
            pltpu.VMEM((N,), jnp.int32),
            pltpu.VMEM((N,), jnp.int32),
            pltpu.VMEM((N,), jnp.int32),
            pltpu.VMEM((N,), jnp.int32),
            pltpu.VMEM((N,), jnp.int32),
            pltpu.VMEM((N,), jnp.int32),
            pltpu.VMEM((N,), jnp.int32),
            pltpu.VMEM((N,), jnp.int32),
            pltpu.VMEM((N,), jnp.int32),
            pltpu.VMEM((N,), jnp.int32),
            pltpu.VMEM((N,), jnp.int32),
            pltpu.VMEM((CHUNK,), jnp.int32),
            pltpu.VMEM((CHUNK,), jnp.int32),
            pltpu.VMEM((CHUNK,), jnp.int32),
            pltpu.VMEM((CHUNK,), jnp.int32),
            pltpu.SemaphoreType.DMA((2, 2)),
        ],
    )
    agg_parts, counts = sc(t_pack, edges[0], edges[1])

    out = pl.pallas_call(
        _back_body,
        out_shape=jax.ShapeDtypeStruct((N, H), jnp.float32),
    )(h, agg_parts, counts,
      W_ffnn[:H].astype(jnp.float32), Wb[0::2], Wb[1::2],
      b_ffnn.astype(jnp.float32)[None, :])
    return out
